# baseline (speedup 1.0000x reference)
import jax
import jax.numpy as jnp
from jax import lax
from jax.experimental import pallas as pl
from jax.experimental.pallas import tpu as pltpu
from jax.experimental.pallas import tpu_sc as plsc

NUM_EMB = 8192
DIM = 128
B = 64
HW = 1024
N = B * HW

NC = 2
NS = 16
NW = NC * NS

PER_TILE = N // NW
CHUNK = 128
NCHUNK = PER_TILE // CHUNK
NBUF = 2


def _body(q_hbm, w_hbm, out_hbm, idxbuf, rows, shared_w, bounce,
          gsems, dsem, bsem, stsem):
    c = lax.axis_index("c")
    s = lax.axis_index("s")
    wid = s * NC + c
    base = wid * PER_TILE

    ROWS_STAGE = NUM_EMB // NS
    sl = pl.ds(s * ROWS_STAGE, ROWS_STAGE)
    pltpu.async_copy(w_hbm.at[sl, :], shared_w.at[sl, :], stsem)

    pltpu.sync_copy(q_hbm.at[pl.ds(base, PER_TILE)], idxbuf)

    def start_gather(ch, p, src):
        idx = idxbuf.at[pl.ds(ch * CHUNK, CHUNK)]
        pltpu.async_copy(src.at[idx], rows.at[p], gsems.at[p])

    def wait_gather(ch, p):
        idx = idxbuf.at[pl.ds(ch * CHUNK, CHUNK)]
        pltpu.make_async_copy(shared_w.at[idx], rows.at[p], gsems.at[p]).wait()

    def out_slice(ch):
        return out_hbm.at[pl.ds(base + ch * CHUNK, CHUNK), :]

    start_gather(0, 0, w_hbm)
    start_gather(1, 1, w_hbm)
    pltpu.make_async_copy(w_hbm.at[sl, :], shared_w.at[sl, :], stsem).wait()
    plsc.subcore_barrier()

    my_bounce = bounce.at[s]

    for k in range(NCHUNK // 2):
        even, odd = 2 * k, 2 * k + 1
        wait_gather(even, 0)
        pltpu.async_copy(rows.at[0], out_slice(even), dsem)
        wait_gather(odd, 1)
        if k >= 1:
            pltpu.make_async_copy(my_bounce, out_slice(odd - 2), bsem).wait()
        pltpu.sync_copy(rows.at[1], my_bounce)
        pltpu.async_copy(my_bounce, out_slice(odd), bsem)
        if odd + 2 < NCHUNK:
            start_gather(odd + 2, 1, shared_w)
        pltpu.make_async_copy(rows.at[0], out_slice(even), dsem).wait()
        if even + 2 < NCHUNK:
            start_gather(even + 2, 0, shared_w)

    pltpu.make_async_copy(my_bounce, out_slice(NCHUNK - 1), bsem).wait()


@jax.jit
def _lookup(q_flat, w):
    mesh = plsc.VectorSubcoreMesh(core_axis_name="c", subcore_axis_name="s")
    f = pl.kernel(
        _body,
        out_type=jax.ShapeDtypeStruct((N, DIM), jnp.float32),
        mesh=mesh,
        scratch_types=[
            pltpu.VMEM((PER_TILE,), jnp.int32),
            pltpu.VMEM((NBUF, CHUNK, DIM), jnp.float32),
            pltpu.VMEM_SHARED((NUM_EMB, DIM), jnp.float32),
            pltpu.VMEM_SHARED((NS, CHUNK, DIM), jnp.float32),
            pltpu.SemaphoreType.DMA((NBUF,)),
            pltpu.SemaphoreType.DMA,
            pltpu.SemaphoreType.DMA,
            pltpu.SemaphoreType.DMA,
        ],
        compiler_params=pltpu.CompilerParams(
            use_tc_tiling_on_sc=False, needs_layout_passes=False
        ),
    )
    return f(q_flat, w)


def kernel(quantized, embedding_weight):
    q_flat = quantized.reshape(N)
    rows = _lookup(q_flat, embedding_weight)
    emb = rows.reshape(B, 32, 32, DIM).transpose(0, 3, 1, 2)
    return (quantized, emb)


# speedup vs baseline: 1.1761x; 1.1761x over previous
import jax
import jax.numpy as jnp
from jax import lax
from jax.experimental import pallas as pl
from jax.experimental.pallas import tpu as pltpu
from jax.experimental.pallas import tpu_sc as plsc

NUM_EMB = 8192
DIM = 128
B = 64
HW = 1024
N = B * HW

NC = 2
NS = 16
NW = NC * NS

PER_TILE = N // NW
CHUNK = 128
NCHUNK = PER_TILE // CHUNK
NBUF = 3


def _body(q_hbm, w_hbm, out_hbm, idxbuf, rows, shared_w, gsems, ssems):
    c = lax.axis_index("c")
    s = lax.axis_index("s")
    wid = s * NC + c
    base = wid * PER_TILE

    ROWS_STAGE = NUM_EMB // NS
    NSTAGE = ROWS_STAGE // CHUNK

    sl = pl.ds(s * ROWS_STAGE, ROWS_STAGE)
    pltpu.async_copy(w_hbm.at[sl, :], shared_w.at[sl, :], ssems.at[0])

    pltpu.sync_copy(q_hbm.at[pl.ds(base, PER_TILE)], idxbuf)

    def start_gather(ch, p):
        idx = idxbuf.at[pl.ds(ch * CHUNK, CHUNK)]
        pltpu.async_copy(shared_w.at[idx], rows.at[p], gsems.at[p])

    def start_gather_hbm(ch, p):
        idx = idxbuf.at[pl.ds(ch * CHUNK, CHUNK)]
        pltpu.async_copy(w_hbm.at[idx], rows.at[p], gsems.at[p])

    def wait_gather(ch, p):
        idx = idxbuf.at[pl.ds(ch * CHUNK, CHUNK)]
        pltpu.make_async_copy(shared_w.at[idx], rows.at[p], gsems.at[p]).wait()

    def out_slice(ch):
        return out_hbm.at[pl.ds(base + ch * CHUNK, CHUNK), :]

    def start_store(ch, p):
        pltpu.async_copy(rows.at[p], out_slice(ch), ssems.at[p])

    def wait_store(ch, p):
        pltpu.make_async_copy(rows.at[p], out_slice(ch), ssems.at[p]).wait()

    DELAY = 2
    for ch in range(min(NBUF, NCHUNK)):
        start_gather_hbm(ch, ch % NBUF)
    pltpu.make_async_copy(w_hbm.at[sl, :], shared_w.at[sl, :], ssems.at[0]).wait()
    plsc.subcore_barrier()

    def chunk_body(ch, carry):
        p = lax.rem(ch, NBUF)
        wait_gather(ch, p)
        start_store(ch, p)
        d = ch - DELAY

        @pl.when(d >= 0)
        def _():
            dp = lax.rem(d, NBUF)
            wait_store(d, dp)
            start_gather(d + NBUF, dp)

        return carry

    lax.fori_loop(0, NCHUNK - NBUF + DELAY, chunk_body, 0)
    for ch in range(NCHUNK - NBUF + DELAY, NCHUNK):
        p = ch % NBUF
        wait_gather(ch, p)
        start_store(ch, p)
    for ch in range(NCHUNK - NBUF, NCHUNK):
        wait_store(ch, ch % NBUF)


@jax.jit
def _lookup(q_flat, w):
    mesh = plsc.VectorSubcoreMesh(core_axis_name="c", subcore_axis_name="s")
    f = pl.kernel(
        _body,
        out_type=jax.ShapeDtypeStruct((N, DIM), jnp.float32),
        mesh=mesh,
        scratch_types=[
            pltpu.VMEM((PER_TILE,), jnp.int32),
            pltpu.VMEM((NBUF, CHUNK, DIM), jnp.float32),
            pltpu.VMEM_SHARED((NUM_EMB, DIM), jnp.float32),
            pltpu.SemaphoreType.DMA((NBUF,)),
            pltpu.SemaphoreType.DMA((NBUF,)),
        ],
        compiler_params=pltpu.CompilerParams(
            use_tc_tiling_on_sc=False, needs_layout_passes=False
        ),
    )
    return f(q_flat, w)


def kernel(quantized, embedding_weight):
    q_flat = quantized.reshape(N)
    rows = _lookup(q_flat, embedding_weight)
    emb = rows.reshape(B, 32, 32, DIM).transpose(0, 3, 1, 2)
    return (quantized, emb)
